# baseline (speedup 1.0000x reference)
import functools

import jax
import jax.numpy as jnp
from jax import lax
from jax.experimental import pallas as pl
from jax.experimental.pallas import tpu as pltpu
from jax.experimental.pallas import tpu_sc as plsc

N_NODES = 10000
N_EDGES = 320000
D = 128
DH = D // 2
NC = 2
NS = 16
EPW = N_EDGES // NS
CHUNK = 128
NCHUNK = 160
HALF_CHUNKS = NCHUNK // 2
GROUP = 5
CNT_W = 16
NP = 10240
PAD_IDX = NP - 1
ROWS_PER_SUB = NP // NS
CNT_ROWS = 2 * NP
CNT_PER_SUB = CNT_ROWS // NS


def _sc_aggregate(x_cols, src_all, dst2d, zeros_agg, zeros_cnt, ones_rows):
    mesh = plsc.VectorSubcoreMesh(core_axis_name="c", subcore_axis_name="s")

    @functools.partial(
        pl.kernel,
        out_type=(
            jax.ShapeDtypeStruct((NC * NP, DH), jnp.float32),
            jax.ShapeDtypeStruct((NC * CNT_ROWS, CNT_W), jnp.float32),
        ),
        mesh=mesh,
        scratch_types=[
            pltpu.VMEM((2, GROUP, CHUNK), jnp.int32),
            pltpu.VMEM((2, GROUP, CHUNK), jnp.int32),
            pltpu.VMEM((GROUP, CHUNK, DH), jnp.float32),
            pltpu.VMEM((CHUNK, CNT_W), jnp.float32),
            pltpu.VMEM_SHARED((NP, DH), jnp.float32),
            pltpu.VMEM_SHARED((CNT_ROWS, CNT_W), jnp.float32),
            pltpu.SemaphoreType.DMA,
            pltpu.SemaphoreType.DMA,
        ],
        compiler_params=pltpu.CompilerParams(use_tc_tiling_on_sc=False),
    )
    def k(x_hbm, src_hbm, dst_hbm, zagg_hbm, zcnt_hbm, ones_hbm,
          agg_out, cnt_out,
          sidx, didx, rows, ones_v, agg_sh, cnt_sh, gsem, isem):
        cid = lax.axis_index("c")
        sid = lax.axis_index("s")
        wid = cid * NS + sid

        rbase = sid * ROWS_PER_SUB
        cbase = sid * CNT_PER_SUB
        init = [
            pltpu.async_copy(ones_hbm, ones_v, gsem),
            pltpu.async_copy(src_hbm.at[wid, pl.ds(0, GROUP)], sidx.at[0],
                             gsem),
            pltpu.async_copy(dst_hbm.at[sid, pl.ds(0, GROUP)], didx.at[0],
                             gsem),
        ]
        pltpu.sync_copy(zagg_hbm.at[pl.ds(rbase, ROWS_PER_SUB)],
                        agg_sh.at[pl.ds(rbase, ROWS_PER_SUB)])
        pltpu.sync_copy(zcnt_hbm.at[pl.ds(cbase, CNT_PER_SUB)],
                        cnt_sh.at[pl.ds(cbase, CNT_PER_SUB)])
        for cp in init:
            cp.wait()
        plsc.subcore_barrier()

        def process_half(j, p, icps):
            gcps = [pltpu.async_copy(
                x_hbm.at[sidx.at[p, b]], rows.at[b], gsem)
                for b in range(GROUP)]

            @pl.when((j < HALF_CHUNKS) == (cid == 0))
            def _():
                for b in range(GROUP):
                    pltpu.sync_copy(ones_v, cnt_sh.at[sidx.at[p, b]],
                                    add=True)

            for b in range(GROUP):
                gcps[b].wait()
                pltpu.sync_copy(rows.at[b], agg_sh.at[didx.at[p, b]],
                                add=True)
            for cp in icps:
                cp.wait()

        def load_idx(j, p):
            return [
                pltpu.async_copy(src_hbm.at[wid, pl.ds(j, GROUP)],
                                 sidx.at[p], isem),
                pltpu.async_copy(dst_hbm.at[sid, pl.ds(j, GROUP)],
                                 didx.at[p], isem),
            ]

        @pl.loop(0, NCHUNK, step=2 * GROUP)
        def _(j):
            icps1 = load_idx(j + GROUP, 1)
            process_half(j, 0, icps1)
            nxt = jnp.minimum(j + 2 * GROUP, NCHUNK - GROUP)
            icps0 = load_idx(nxt, 0)
            process_half(j + GROUP, 1, icps0)

        plsc.subcore_barrier()
        pltpu.sync_copy(agg_sh.at[pl.ds(rbase, ROWS_PER_SUB)],
                        agg_out.at[pl.ds(cid * NP + rbase, ROWS_PER_SUB)])
        pltpu.sync_copy(cnt_sh.at[pl.ds(cbase, CNT_PER_SUB)],
                        cnt_out.at[pl.ds(cid * CNT_ROWS + cbase, CNT_PER_SUB)])

    return k(x_cols, src_all, dst2d, zeros_agg, zeros_cnt, ones_rows)


def _tc_dense_body(x_ref, agg_ref, cnt_ref, w1_ref, b1_ref, w2_ref, b2_ref,
                   out_ref):
    w1 = w1_ref[...]
    a0 = agg_ref[:N_NODES, :]
    a1 = agg_ref[NP:NP + N_NODES, :]
    z = (jnp.dot(x_ref[...], w1, preferred_element_type=jnp.float32)
         + jnp.dot(a0, w1[:DH, :], preferred_element_type=jnp.float32)
         + jnp.dot(a1, w1[DH:, :], preferred_element_type=jnp.float32)
         + b1_ref[...])
    r = jnp.maximum(z, 0.0)
    w = (1.0 + cnt_ref[:N_NODES, 0:1]
         + cnt_ref[CNT_ROWS + NP:CNT_ROWS + NP + N_NODES, 0:1])
    s = jnp.sum(r * w, axis=0, keepdims=True)
    out_ref[...] = (jnp.dot(s, w2_ref[...], preferred_element_type=jnp.float32)
                    + float(N_NODES) * b2_ref[...])


def _tc_dense(feats, agg, cnt, W1, b1, W2, b2):
    return pl.pallas_call(
        _tc_dense_body,
        out_shape=jax.ShapeDtypeStruct((1, D), jnp.float32),
    )(feats, agg, cnt, W1, b1.reshape(1, D), W2, b2.reshape(1, D))


def kernel(feats, edge_index, W1, b1, W2, b2):
    ei = edge_index.astype(jnp.int32)
    pad = ((0, 0), (0, NCHUNK * CHUNK - EPW))
    srcp = jnp.pad(ei[0].reshape(NS, EPW), pad,
                   constant_values=PAD_IDX).reshape(NS, NCHUNK, CHUNK)
    dstp = jnp.pad(ei[1].reshape(NS, EPW), pad,
                   constant_values=PAD_IDX).reshape(NS, NCHUNK, CHUNK)
    src_all = jnp.concatenate([srcp, srcp + NP], axis=0)
    rowpad = ((0, NP - N_NODES), (0, 0))
    x_cols = jnp.concatenate([jnp.pad(feats[:, :DH], rowpad),
                              jnp.pad(feats[:, DH:], rowpad)], axis=0)
    zeros_agg = jnp.zeros((NP, DH), jnp.float32)
    zeros_cnt = jnp.zeros((CNT_ROWS, CNT_W), jnp.float32)
    ones_rows = jnp.zeros((CHUNK, CNT_W), jnp.float32).at[:, 0].set(1.0)
    agg, cnt = _sc_aggregate(x_cols, src_all, dstp, zeros_agg, zeros_cnt,
                             ones_rows)
    return _tc_dense(feats, agg, cnt, W1, b1, W2, b2)
